# baseline (speedup 1.0000x reference)
import jax
import jax.numpy as jnp
from jax import lax
from jax.experimental import pallas as pl

_B, _LQ, _D = 32, 32, 1024
_NF, _OPF = 64, 32
_NOCR = _NF * _OPF
_FRAME_TOPK, _OCR_TOPK = 8, 4
_M = _FRAME_TOPK * _OCR_TOPK
_RSQRT_D = 1.0 / 32.0


def _body(qf_ref, qmask_ref, ff_ref, fmask_ref, of_ref, omask_ref,
          obox_ref, tid_ref, Wq_ref, bq_ref, wsa_ref, bsa_ref,
          Wt_ref, Ws_ref,
          gf_out, box_out, gfm_out, nfm_out, gom_out, nom_out):
    f32 = jnp.float32

    qf = qf_ref[0]
    qp = jnp.dot(qf, Wq_ref[...], preferred_element_type=f32) + bq_ref[...]
    a = jnp.dot(qp.astype(jnp.bfloat16), wsa_ref[...].astype(jnp.bfloat16),
                preferred_element_type=f32) + bsa_ref[0, 0]
    m = jnp.max(a, axis=0, keepdims=True)
    e = jnp.exp(a - m)
    p = e / jnp.sum(e, axis=0, keepdims=True)
    p = p * qmask_ref[0]
    p = p / (jnp.sum(p, axis=0, keepdims=True) + 1e-12)
    gq = lax.dot_general(p.astype(jnp.bfloat16), qp.astype(jnp.bfloat16),
                         (((0,), (0,)), ((), ())),
                         preferred_element_type=f32)

    qt = jnp.dot(gq, Wt_ref[...], preferred_element_type=f32)
    qs = jnp.dot(gq, Ws_ref[...], preferred_element_type=f32)

    ff = ff_ref[0]
    sf = lax.dot_general(qt, ff, (((1,), (1,)), ((), ())),
                         preferred_element_type=f32) * _RSQRT_D
    fmask = fmask_ref[0]
    sf = jnp.where(fmask > 0, sf, -1e9)

    iota_f = lax.broadcasted_iota(jnp.int32, (1, _NF), 1)
    iota_k = lax.broadcasted_iota(jnp.int32, (1, _FRAME_TOPK), 1)
    tid = tid_ref[0]
    gf_row = jnp.zeros((1, _FRAME_TOPK), jnp.int32)
    sel_f = jnp.zeros((1, _NF), f32)
    tmatch = jnp.zeros((1, _NOCR), jnp.bool_)
    v = sf
    for k in range(_FRAME_TOPK):
        mv = jnp.max(v, axis=1, keepdims=True)
        i = jnp.min(jnp.where(v == mv, iota_f, _NF), axis=1, keepdims=True)
        gf_row = jnp.where(iota_k == k, i + 1, gf_row)
        sel_f = jnp.where(iota_f == i, 1.0, sel_f)
        tmatch = jnp.logical_or(tmatch, tid == (i + 1))
        v = jnp.where(iota_f == i, -jnp.inf, v)
    gf_out[0] = gf_row
    gfm_out[0] = sel_f * fmask
    nfm_out[0] = (1.0 - sel_f) * fmask

    omask = omask_ref[0]
    valid = omask * tmatch.astype(f32)
    of = of_ref[0]
    so = lax.dot_general(qs, of, (((1,), (1,)), ((), ())),
                         preferred_element_type=f32) * _RSQRT_D
    so = jnp.where(valid > 0, so, -1e9)

    iota_o = lax.broadcasted_iota(jnp.int32, (1, _NOCR), 1)
    rows = []
    sel_o = jnp.zeros((1, _NOCR), f32)
    v = so
    for k in range(_M):
        mv = jnp.max(v, axis=1, keepdims=True)
        i = jnp.min(jnp.where(v == mv, iota_o, _NOCR), axis=1, keepdims=True)
        oh = (iota_o == i).astype(f32)
        rows.append(oh)
        sel_o = jnp.maximum(sel_o, oh)
        v = jnp.where(iota_o == i, -jnp.inf, v)
    onehot = jnp.concatenate(rows, axis=0)
    box_out[0] = jnp.dot(onehot, obox_ref[0], preferred_element_type=f32, precision=lax.Precision.HIGHEST)
    go = sel_o * valid
    gom_out[0] = go
    nom_out[0] = (1.0 - go) * valid


def kernel(q_feat, q_mask, frame_feat, frame_mask, ocr_feat, ocr_mask,
           ocr_box, temporal_id, Wq, bq, w_sa, b_sa, Wt, Ws):
    B, LQ, D = _B, _LQ, _D

    qmask_c = q_mask.reshape(B, LQ, 1)
    fmask_r = frame_mask.reshape(B, 1, _NF)
    omask_r = ocr_mask.reshape(B, 1, _NOCR)
    tid_r = temporal_id.astype(jnp.int32).reshape(B, 1, _NOCR)
    bq2 = bq.reshape(1, D)
    wsa2 = w_sa.reshape(D, 1)
    bsa2 = b_sa.reshape(1, 1).astype(jnp.float32)

    def bmap(b):
        return (b, 0, 0)

    def wmap2(b):
        return (0, 0)

    in_specs = [
        pl.BlockSpec((1, LQ, D), bmap),
        pl.BlockSpec((1, LQ, 1), bmap),
        pl.BlockSpec((1, _NF, D), bmap),
        pl.BlockSpec((1, 1, _NF), bmap),
        pl.BlockSpec((1, _NOCR, D), bmap),
        pl.BlockSpec((1, 1, _NOCR), bmap),
        pl.BlockSpec((1, _NOCR, 4), bmap),
        pl.BlockSpec((1, 1, _NOCR), bmap),
        pl.BlockSpec((D, D), wmap2),
        pl.BlockSpec((1, D), wmap2),
        pl.BlockSpec((D, 1), wmap2),
        pl.BlockSpec((1, 1), wmap2),
        pl.BlockSpec((D, D), wmap2),
        pl.BlockSpec((D, D), wmap2),
    ]
    out_shape = [
        jax.ShapeDtypeStruct((B, 1, _FRAME_TOPK), jnp.int32),
        jax.ShapeDtypeStruct((B, _M, 4), jnp.float32),
        jax.ShapeDtypeStruct((B, 1, _NF), jnp.float32),
        jax.ShapeDtypeStruct((B, 1, _NF), jnp.float32),
        jax.ShapeDtypeStruct((B, 1, _NOCR), jnp.float32),
        jax.ShapeDtypeStruct((B, 1, _NOCR), jnp.float32),
    ]
    out_specs = [
        pl.BlockSpec((1, 1, _FRAME_TOPK), bmap),
        pl.BlockSpec((1, _M, 4), bmap),
        pl.BlockSpec((1, 1, _NF), bmap),
        pl.BlockSpec((1, 1, _NF), bmap),
        pl.BlockSpec((1, 1, _NOCR), bmap),
        pl.BlockSpec((1, 1, _NOCR), bmap),
    ]

    gf, box, gfm, nfm, gom, nom = pl.pallas_call(
        _body,
        grid=(B,),
        in_specs=in_specs,
        out_specs=out_specs,
        out_shape=out_shape,
    )(q_feat, qmask_c, frame_feat, fmask_r, ocr_feat, omask_r,
      ocr_box, tid_r, Wq, bq2, wsa2, bsa2, Wt, Ws)

    return (gf.reshape(B, _FRAME_TOPK), box, gfm.reshape(B, _NF),
            nfm.reshape(B, _NF), gom.reshape(B, _NOCR),
            nom.reshape(B, _NOCR))


# speedup vs baseline: 1.0068x; 1.0068x over previous
import jax
import jax.numpy as jnp
from jax import lax
from jax.experimental import pallas as pl

_B, _LQ, _D = 32, 32, 1024
_NF, _OPF = 64, 32
_NOCR = _NF * _OPF
_FRAME_TOPK, _OCR_TOPK = 8, 4
_M = _FRAME_TOPK * _OCR_TOPK
_RSQRT_D = 1.0 / 32.0


def _body(qf_ref, qmask_ref, ff_ref, fmask_ref, of_ref, omask_ref,
          obox_ref, tid_ref, Wq_ref, bq_ref, wsa_ref, bsa_ref,
          Wt_ref, Ws_ref,
          gf_out, box_out, gfm_out, nfm_out, gom_out, nom_out):
    f32 = jnp.float32

    qf = qf_ref[0]
    qp = jnp.dot(qf, Wq_ref[...], preferred_element_type=f32) + bq_ref[...]
    a = jnp.dot(qp.astype(jnp.bfloat16), wsa_ref[...].astype(jnp.bfloat16),
                preferred_element_type=f32) + bsa_ref[0, 0]
    m = jnp.max(a, axis=0, keepdims=True)
    e = jnp.exp(a - m)
    p = e / jnp.sum(e, axis=0, keepdims=True)
    p = p * qmask_ref[0]
    p = p / (jnp.sum(p, axis=0, keepdims=True) + 1e-12)
    gq = lax.dot_general(p.astype(jnp.bfloat16), qp.astype(jnp.bfloat16),
                         (((0,), (0,)), ((), ())),
                         preferred_element_type=f32)

    qt = jnp.dot(gq, Wt_ref[...], preferred_element_type=f32)
    qs = jnp.dot(gq, Ws_ref[...], preferred_element_type=f32)

    ff = ff_ref[0]
    sf = lax.dot_general(qt, ff, (((1,), (1,)), ((), ())),
                         preferred_element_type=f32) * _RSQRT_D
    fmask = fmask_ref[0]
    sf = jnp.where(fmask > 0, sf, -1e9)

    iota_f = lax.broadcasted_iota(jnp.int32, (1, _NF), 1)
    iota_k = lax.broadcasted_iota(jnp.int32, (1, _FRAME_TOPK), 1)
    tid = tid_ref[0]
    gf_row = jnp.zeros((1, _FRAME_TOPK), jnp.int32)
    tmatch = jnp.zeros((1, _NOCR), jnp.bool_)
    frows = []
    v = sf
    for k in range(_FRAME_TOPK):
        mv = jnp.max(v, axis=1, keepdims=True)
        i = jnp.min(jnp.where(v == mv, iota_f, _NF), axis=1, keepdims=True)
        gf_row = jnp.where(iota_k == k, i + 1, gf_row)
        cmp = iota_f == i
        frows.append(cmp.astype(f32))
        tmatch = jnp.logical_or(tmatch, tid == (i + 1))
        v = jnp.where(cmp, -jnp.inf, v)
    sel_f = jnp.clip(jnp.sum(jnp.concatenate(frows, axis=0), axis=0,
                             keepdims=True), 0.0, 1.0)
    gf_out[0] = gf_row
    gfm_out[0] = sel_f * fmask
    nfm_out[0] = (1.0 - sel_f) * fmask

    omask = omask_ref[0]
    valid = omask * tmatch.astype(f32)
    of = of_ref[0]
    so = lax.dot_general(qs, of, (((1,), (1,)), ((), ())),
                         preferred_element_type=f32) * _RSQRT_D
    so = jnp.where(valid > 0, so, -1e9)

    iota_o = lax.broadcasted_iota(jnp.int32, (1, _NOCR), 1)
    rows = []
    v = so
    for k in range(_M):
        mv = jnp.max(v, axis=1, keepdims=True)
        i = jnp.min(jnp.where(v == mv, iota_o, _NOCR), axis=1, keepdims=True)
        cmp = iota_o == i
        rows.append(cmp.astype(f32))
        v = jnp.where(cmp, -jnp.inf, v)
    onehot = jnp.concatenate(rows, axis=0)
    sel_o = jnp.clip(jnp.sum(onehot, axis=0, keepdims=True), 0.0, 1.0)
    box_out[0] = jnp.dot(onehot, obox_ref[0], preferred_element_type=f32, precision=lax.Precision.HIGHEST)
    go = sel_o * valid
    gom_out[0] = go
    nom_out[0] = (1.0 - go) * valid


def kernel(q_feat, q_mask, frame_feat, frame_mask, ocr_feat, ocr_mask,
           ocr_box, temporal_id, Wq, bq, w_sa, b_sa, Wt, Ws):
    B, LQ, D = _B, _LQ, _D

    qmask_c = q_mask.reshape(B, LQ, 1)
    fmask_r = frame_mask.reshape(B, 1, _NF)
    omask_r = ocr_mask.reshape(B, 1, _NOCR)
    tid_r = temporal_id.astype(jnp.int32).reshape(B, 1, _NOCR)
    bq2 = bq.reshape(1, D)
    wsa2 = w_sa.reshape(D, 1)
    bsa2 = b_sa.reshape(1, 1).astype(jnp.float32)

    def bmap(b):
        return (b, 0, 0)

    def wmap2(b):
        return (0, 0)

    in_specs = [
        pl.BlockSpec((1, LQ, D), bmap),
        pl.BlockSpec((1, LQ, 1), bmap),
        pl.BlockSpec((1, _NF, D), bmap),
        pl.BlockSpec((1, 1, _NF), bmap),
        pl.BlockSpec((1, _NOCR, D), bmap),
        pl.BlockSpec((1, 1, _NOCR), bmap),
        pl.BlockSpec((1, _NOCR, 4), bmap),
        pl.BlockSpec((1, 1, _NOCR), bmap),
        pl.BlockSpec((D, D), wmap2),
        pl.BlockSpec((1, D), wmap2),
        pl.BlockSpec((D, 1), wmap2),
        pl.BlockSpec((1, 1), wmap2),
        pl.BlockSpec((D, D), wmap2),
        pl.BlockSpec((D, D), wmap2),
    ]
    out_shape = [
        jax.ShapeDtypeStruct((B, 1, _FRAME_TOPK), jnp.int32),
        jax.ShapeDtypeStruct((B, _M, 4), jnp.float32),
        jax.ShapeDtypeStruct((B, 1, _NF), jnp.float32),
        jax.ShapeDtypeStruct((B, 1, _NF), jnp.float32),
        jax.ShapeDtypeStruct((B, 1, _NOCR), jnp.float32),
        jax.ShapeDtypeStruct((B, 1, _NOCR), jnp.float32),
    ]
    out_specs = [
        pl.BlockSpec((1, 1, _FRAME_TOPK), bmap),
        pl.BlockSpec((1, _M, 4), bmap),
        pl.BlockSpec((1, 1, _NF), bmap),
        pl.BlockSpec((1, 1, _NF), bmap),
        pl.BlockSpec((1, 1, _NOCR), bmap),
        pl.BlockSpec((1, 1, _NOCR), bmap),
    ]

    gf, box, gfm, nfm, gom, nom = pl.pallas_call(
        _body,
        grid=(B,),
        in_specs=in_specs,
        out_specs=out_specs,
        out_shape=out_shape,
    )(q_feat, qmask_c, frame_feat, fmask_r, ocr_feat, omask_r,
      ocr_box, tid_r, Wq, bq2, wsa2, bsa2, Wt, Ws)

    return (gf.reshape(B, _FRAME_TOPK), box, gfm.reshape(B, _NF),
            nfm.reshape(B, _NF), gom.reshape(B, _NOCR),
            nom.reshape(B, _NOCR))
